# baseline (speedup 1.0000x reference)
import math

import jax
import jax.numpy as jnp
from jax import lax
from jax.experimental import pallas as pl
from jax.experimental.pallas import tpu as pltpu
from jax.experimental.pallas import tpu_sc as plsc

RESOLUTION = 1024
TWO_PI = 2.0 * math.pi

NC = 2
NS = 16
L = 16
NW = NC * NS

N_ROWS = 16384
N_COLS = 200
NSPLIT = 2
ROWS_CALL = N_ROWS // NSPLIT
ROWS_PER_W = ROWS_CALL // NW
ROWS_PER_CHUNK = 32
NCH = ROWS_PER_W // ROWS_PER_CHUNK
NB = 4

COL_OFFS = tuple(16 * j for j in range(N_COLS // L)) + (N_COLS - L,)


def _phase_body(ang_hbm, sin_t_hbm, cos_t_hbm, sin_out_hbm, cos_out_hbm,
                *scratch):
    ang_v = scratch[0:NB]
    sin_v = scratch[NB:2 * NB]
    cos_v = scratch[2 * NB:3 * NB]
    sin_tab, cos_tab = scratch[3 * NB], scratch[3 * NB + 1]
    in_sems = scratch[3 * NB + 2:3 * NB + 2 + NB]
    s_sems = scratch[3 * NB + 2 + NB:3 * NB + 2 + 2 * NB]
    c_sems = scratch[3 * NB + 2 + 2 * NB:3 * NB + 2 + 3 * NB]

    wid = lax.axis_index("s") * NC + lax.axis_index("c")
    row_base = wid * ROWS_PER_W
    scale = jnp.float32((RESOLUTION - 1) / TWO_PI)

    pltpu.sync_copy(sin_t_hbm, sin_tab)
    pltpu.sync_copy(cos_t_hbm, cos_tab)

    in_cp = [None] * NB
    s_cp = [None] * NB
    c_cp = [None] * NB

    for b in range(NB):
        r0 = row_base + b * ROWS_PER_CHUNK
        in_cp[b] = pltpu.async_copy(
            ang_hbm.at[pl.ds(r0, ROWS_PER_CHUNK), :], ang_v[b], in_sems[b])

    for c in range(NCH):
        b = c % NB
        in_cp[b].wait()
        if s_cp[b] is not None:
            s_cp[b].wait()
            c_cp[b].wait()

        av, sv, cv = ang_v[b], sin_v[b], cos_v[b]

        @plsc.parallel_loop(0, ROWS_PER_CHUNK)
        def _(r):
            for off in COL_OFFS:
                sl = pl.ds(off, L)
                a = av[r, sl]
                idx = (a * scale).astype(jnp.int32)
                sv[r, sl] = plsc.load_gather(sin_tab, [idx])
                cv[r, sl] = plsc.load_gather(cos_tab, [idx])

        r0 = row_base + c * ROWS_PER_CHUNK
        s_cp[b] = pltpu.async_copy(
            sin_v[b], sin_out_hbm.at[pl.ds(r0, ROWS_PER_CHUNK), :], s_sems[b])
        c_cp[b] = pltpu.async_copy(
            cos_v[b], cos_out_hbm.at[pl.ds(r0, ROWS_PER_CHUNK), :], c_sems[b])

        nxt = c + NB
        if nxt < NCH:
            nr0 = row_base + nxt * ROWS_PER_CHUNK
            in_cp[b] = pltpu.async_copy(
                ang_hbm.at[pl.ds(nr0, ROWS_PER_CHUNK), :], ang_v[b],
                in_sems[b])

    for b in range(NB):
        s_cp[b].wait()
        c_cp[b].wait()


@jax.jit
def kernel(angles, sin_table, cos_table):
    mesh = plsc.VectorSubcoreMesh(core_axis_name="c", subcore_axis_name="s")
    run = pl.kernel(
        _phase_body,
        out_type=(
            jax.ShapeDtypeStruct((ROWS_CALL, N_COLS), jnp.float32),
            jax.ShapeDtypeStruct((ROWS_CALL, N_COLS), jnp.float32),
        ),
        mesh=mesh,
        scratch_types=(
            [pltpu.VMEM((ROWS_PER_CHUNK, N_COLS), jnp.float32)
             for _ in range(3 * NB)]
            + [pltpu.VMEM((RESOLUTION,), jnp.float32) for _ in range(2)]
            + [pltpu.SemaphoreType.DMA for _ in range(3 * NB)]
        ),
        compiler_params=pltpu.CompilerParams(
            needs_layout_passes=False, use_tc_tiling_on_sc=True),
    )
    outs = [
        run(lax.slice_in_dim(angles, i * ROWS_CALL, (i + 1) * ROWS_CALL,
                             axis=0),
            sin_table, cos_table)
        for i in range(NSPLIT)
    ]
    sin_vals = jnp.concatenate([o[0] for o in outs], axis=0)
    cos_vals = jnp.concatenate([o[1] for o in outs], axis=0)
    return sin_vals, cos_vals


# speedup vs baseline: 1.5897x; 1.5897x over previous
import math

import jax
import jax.numpy as jnp
from jax import lax
from jax.experimental import pallas as pl
from jax.experimental.pallas import tpu as pltpu
from jax.experimental.pallas import tpu_sc as plsc

RESOLUTION = 1024
TWO_PI = 2.0 * math.pi

NC = 2
NS = 16
L = 16
NW = NC * NS

N_ROWS = 16384
N_COLS = 200
ROWS_PER_W = N_ROWS // NW
ROWS_PER_CHUNK = 32
NCH = ROWS_PER_W // ROWS_PER_CHUNK
NB = 4

COL_OFFS = tuple(16 * j for j in range(N_COLS // L)) + (N_COLS - L,)


def _phase_body(ang_hbm, sin_t_hbm, cos_t_hbm, sin_out_hbm, cos_out_hbm,
                *scratch):
    ang_v = scratch[0:NB]
    sin_v = scratch[NB:2 * NB]
    cos_v = scratch[2 * NB:3 * NB]
    sin_tab, cos_tab = scratch[3 * NB], scratch[3 * NB + 1]
    in_sems = scratch[3 * NB + 2:3 * NB + 2 + NB]
    s_sems = scratch[3 * NB + 2 + NB:3 * NB + 2 + 2 * NB]
    c_sems = scratch[3 * NB + 2 + 2 * NB:3 * NB + 2 + 3 * NB]
    tab_sems = scratch[3 * NB + 2 + 3 * NB:3 * NB + 4 + 3 * NB]

    wid = lax.axis_index("s") * NC + lax.axis_index("c")
    row_base = wid * ROWS_PER_W
    scale = jnp.float32((RESOLUTION - 1) / TWO_PI)
    rpc = ROWS_PER_CHUNK

    st_cp = pltpu.async_copy(sin_t_hbm, sin_tab, tab_sems[0])
    ct_cp = pltpu.async_copy(cos_t_hbm, cos_tab, tab_sems[1])
    for b in range(NB):
        pltpu.async_copy(
            ang_hbm.at[pl.ds(row_base + b * rpc, rpc), :], ang_v[b],
            in_sems[b])
    st_cp.wait()
    ct_cp.wait()

    @pl.loop(0, NCH, step=NB)
    def _round(g):
        for b in range(NB):
            c = g + b
            r0 = row_base + c * rpc
            pltpu.make_async_copy(
                ang_hbm.at[pl.ds(r0, rpc), :], ang_v[b], in_sems[b]).wait()

            @pl.when(g > 0)
            def _():
                pltpu.make_async_copy(
                    sin_v[b], sin_out_hbm.at[pl.ds(r0, rpc), :],
                    s_sems[b]).wait()
                pltpu.make_async_copy(
                    cos_v[b], cos_out_hbm.at[pl.ds(r0, rpc), :],
                    c_sems[b]).wait()

            av, sv, cv = ang_v[b], sin_v[b], cos_v[b]

            @plsc.parallel_loop(0, ROWS_PER_CHUNK)
            def _(r):
                for off in COL_OFFS:
                    sl = pl.ds(off, L)
                    a = av[r, sl]
                    idx = (a * scale).astype(jnp.int32)
                    sv[r, sl] = plsc.load_gather(sin_tab, [idx])
                    cv[r, sl] = plsc.load_gather(cos_tab, [idx])

            pltpu.async_copy(
                sin_v[b], sin_out_hbm.at[pl.ds(r0, rpc), :], s_sems[b])
            pltpu.async_copy(
                cos_v[b], cos_out_hbm.at[pl.ds(r0, rpc), :], c_sems[b])

            @pl.when(c + NB < NCH)
            def _():
                nr0 = row_base + (c + NB) * rpc
                pltpu.async_copy(
                    ang_hbm.at[pl.ds(nr0, rpc), :], ang_v[b], in_sems[b])

    for b in range(NB):
        r0 = row_base + (NCH - NB + b) * rpc
        pltpu.make_async_copy(
            sin_v[b], sin_out_hbm.at[pl.ds(r0, rpc), :], s_sems[b]).wait()
        pltpu.make_async_copy(
            cos_v[b], cos_out_hbm.at[pl.ds(r0, rpc), :], c_sems[b]).wait()


@jax.jit
def kernel(angles, sin_table, cos_table):
    mesh = plsc.VectorSubcoreMesh(core_axis_name="c", subcore_axis_name="s")
    run = pl.kernel(
        _phase_body,
        out_type=(
            jax.ShapeDtypeStruct((N_ROWS, N_COLS), jnp.float32),
            jax.ShapeDtypeStruct((N_ROWS, N_COLS), jnp.float32),
        ),
        mesh=mesh,
        scratch_types=(
            [pltpu.VMEM((ROWS_PER_CHUNK, N_COLS), jnp.float32)
             for _ in range(3 * NB)]
            + [pltpu.VMEM((RESOLUTION,), jnp.float32) for _ in range(2)]
            + [pltpu.SemaphoreType.DMA for _ in range(3 * NB + 2)]
        ),
        compiler_params=pltpu.CompilerParams(
            needs_layout_passes=False, use_tc_tiling_on_sc=True),
    )
    return run(angles, sin_table, cos_table)


# speedup vs baseline: 1.5954x; 1.0036x over previous
import math

import jax
import jax.numpy as jnp
from jax import lax
from jax.experimental import pallas as pl
from jax.experimental.pallas import tpu as pltpu
from jax.experimental.pallas import tpu_sc as plsc

RESOLUTION = 1024
TWO_PI = 2.0 * math.pi

NC = 2
NS = 16
L = 16
NW = NC * NS

N_ROWS = 16384
N_COLS = 200
ROWS_PER_W = N_ROWS // NW
ROWS_PER_CHUNK = 32
NCH = ROWS_PER_W // ROWS_PER_CHUNK
NB = 4

COL_OFFS = tuple(16 * j for j in range(N_COLS // L)) + (N_COLS - L,)


def _phase_body(ang_hbm, sin_t_hbm, cos_t_hbm, sin_out_hbm, cos_out_hbm,
                *scratch):
    ang_v = scratch[0:NB]
    sin_v = scratch[NB:2 * NB]
    cos_v = scratch[2 * NB:3 * NB]
    sin_tab, cos_tab = scratch[3 * NB], scratch[3 * NB + 1]
    in_sems = scratch[3 * NB + 2:3 * NB + 2 + NB]
    s_sems = scratch[3 * NB + 2 + NB:3 * NB + 2 + 2 * NB]
    c_sems = scratch[3 * NB + 2 + 2 * NB:3 * NB + 2 + 3 * NB]
    tab_sems = scratch[3 * NB + 2 + 3 * NB:3 * NB + 4 + 3 * NB]

    wid = lax.axis_index("s") * NC + lax.axis_index("c")
    row_base = wid * ROWS_PER_W
    scale = jnp.float32((RESOLUTION - 1) / TWO_PI)
    rpc = ROWS_PER_CHUNK

    st_cp = pltpu.async_copy(sin_t_hbm, sin_tab, tab_sems[0])
    ct_cp = pltpu.async_copy(cos_t_hbm, cos_tab, tab_sems[1])
    for b in range(NB):
        pltpu.async_copy(
            ang_hbm.at[pl.ds(row_base + b * rpc, rpc), :], ang_v[b],
            in_sems[b])
    st_cp.wait()
    ct_cp.wait()

    @pl.loop(0, NCH, step=NB)
    def _round(g):
        for b in range(NB):
            c = g + b
            r0 = row_base + c * rpc
            pltpu.make_async_copy(
                ang_hbm.at[pl.ds(r0, rpc), :], ang_v[b], in_sems[b]).wait()

            @pl.when(g > 0)
            def _():
                pltpu.make_async_copy(
                    sin_v[b], sin_out_hbm.at[pl.ds(r0, rpc), :],
                    s_sems[b]).wait()
                pltpu.make_async_copy(
                    cos_v[b], cos_out_hbm.at[pl.ds(r0, rpc), :],
                    c_sems[b]).wait()

            av, sv, cv = ang_v[b], sin_v[b], cos_v[b]

            @plsc.parallel_loop(0, ROWS_PER_CHUNK)
            def _(r):
                for off in COL_OFFS:
                    sl = pl.ds(off, L)
                    a = av[r, sl]
                    idx = (a * scale).astype(jnp.int32)
                    sv[r, sl] = plsc.load_gather(sin_tab, [idx])
                    cv[r, sl] = plsc.load_gather(cos_tab, [idx])

            pltpu.async_copy(
                sin_v[b], sin_out_hbm.at[pl.ds(r0, rpc), :], s_sems[b])
            pltpu.async_copy(
                cos_v[b], cos_out_hbm.at[pl.ds(r0, rpc), :], c_sems[b])

            @pl.when(c + NB < NCH)
            def _():
                nr0 = row_base + (c + NB) * rpc
                pltpu.async_copy(
                    ang_hbm.at[pl.ds(nr0, rpc), :], ang_v[b], in_sems[b])

    for b in range(NB):
        r0 = row_base + (NCH - NB + b) * rpc
        pltpu.make_async_copy(
            sin_v[b], sin_out_hbm.at[pl.ds(r0, rpc), :], s_sems[b]).wait()
        pltpu.make_async_copy(
            cos_v[b], cos_out_hbm.at[pl.ds(r0, rpc), :], c_sems[b]).wait()


@jax.jit
def kernel(angles, sin_table, cos_table):
    mesh = plsc.VectorSubcoreMesh(core_axis_name="c", subcore_axis_name="s")
    run = pl.kernel(
        _phase_body,
        out_type=(
            jax.ShapeDtypeStruct((N_ROWS, N_COLS), jnp.float32),
            jax.ShapeDtypeStruct((N_ROWS, N_COLS), jnp.float32),
        ),
        mesh=mesh,
        scratch_types=(
            [pltpu.VMEM((ROWS_PER_CHUNK, N_COLS), jnp.float32)
             for _ in range(3 * NB)]
            + [pltpu.VMEM((RESOLUTION,), jnp.float32) for _ in range(2)]
            + [pltpu.SemaphoreType.DMA for _ in range(3 * NB + 2)]
        ),
        compiler_params=pltpu.CompilerParams(
            needs_layout_passes=False, use_tc_tiling_on_sc=True),
    )
    sin_vals, cos_vals = run(angles * jnp.float32(1.0), sin_table, cos_table)
    return sin_vals + jnp.float32(0.0), cos_vals + jnp.float32(0.0)
